# Initial kernel scaffold; baseline (speedup 1.0000x reference)
import jax
import jax.numpy as jnp
from jax.experimental import pallas as pl


def kernel(encoder_input, decoder_input, labels, W_enc, W_dec_z, W_dec_out):
    raise NotImplementedError("write your pallas kernel here")


# speedup vs baseline: 1.3120x; 1.3120x over previous
import jax
import jax.numpy as jnp
from jax.experimental import pallas as pl
from jax.experimental.pallas import tpu as pltpu

_B = 4096
_D = 1024
_L = 64
_K = 8
_C = 1024
_TB = 128
_GRID = _B // _TB


def _fused_step(enc_ref, din_ref, lab_ref, wenc_ref, wdz_ref, wout_ref, out_ref):
    logits = jnp.dot(enc_ref[...], wenc_ref[...],
                     preferred_element_type=jnp.float32)

    iota_l = jax.lax.broadcasted_iota(jnp.int32, (_TB, _L), 1)
    v = logits
    zs, idxs = [], []
    for _ in range(_K):
        m = jnp.max(v, axis=1, keepdims=True)
        i_k = jnp.min(jnp.where(v == m, iota_l, _L), axis=1, keepdims=True)
        v = jnp.where(iota_l == i_k, -1e30, v)
        zs.append(m)
        idxs.append(i_k)

    cs = jnp.zeros((_TB, 1), jnp.float32)
    ksup = jnp.zeros((_TB, 1), jnp.float32)
    cssel = jnp.zeros((_TB, 1), jnp.float32)
    for k in range(_K):
        cs = cs + zs[k]
        sup = ((1.0 + (k + 1) * zs[k]) > cs).astype(jnp.float32)
        ksup = ksup + sup
        cssel = cssel + sup * zs[k]
    tau = (cssel - 1.0) / ksup
    ps = [jnp.maximum(zs[k] - tau, 0.0) for k in range(_K)]
    ent = 0.0
    for k in range(_K):
        ent = ent + jnp.sum(-ps[k] * jnp.log(ps[k] + 1e-10))

    idx_stack = jnp.concatenate(idxs, axis=0)
    p_stack = jnp.concatenate(ps, axis=0)
    onehot = (jax.lax.broadcasted_iota(jnp.int32, (_K * _TB, _L), 1)
              == idx_stack).astype(jnp.bfloat16)
    zrows = jnp.dot(onehot, wdz_ref[...],
                    preferred_element_type=jnp.float32)
    din = din_ref[...]
    din_t = jnp.concatenate([din] * _K, axis=0)
    h = jnp.maximum(zrows + din_t, 0.0).astype(jnp.bfloat16)
    d = jnp.dot(h, wout_ref[...],
                preferred_element_type=jnp.float32)

    m2 = jnp.max(d, axis=1, keepdims=True)
    se = jnp.sum(jnp.exp(d - m2), axis=1, keepdims=True)
    lse = m2 + jnp.log(se)
    lab_t = jnp.concatenate([lab_ref[...]] * _K, axis=0)
    iota_c = jax.lax.broadcasted_iota(jnp.int32, (_K * _TB, _C), 1)
    dlab = jnp.sum(jnp.where(iota_c == lab_t, d, 0.0), axis=1, keepdims=True)
    loss_c = lse - dlab

    partial = jnp.sum(p_stack * loss_c) - 0.01 * ent
    out_ref[...] = jnp.broadcast_to(partial, (1, 1, 128))


def kernel(encoder_input, decoder_input, labels, W_enc, W_dec_z, W_dec_out):
    enc16 = encoder_input.astype(jnp.bfloat16)
    wenc16 = W_enc.astype(jnp.bfloat16)
    wdz16 = W_dec_z.astype(jnp.bfloat16)
    wout16 = W_dec_out.astype(jnp.bfloat16)
    lab2 = labels.astype(jnp.int32).reshape(_B, 1)

    partials = pl.pallas_call(
        _fused_step,
        grid=(_GRID,),
        in_specs=[
            pl.BlockSpec((_TB, _D), lambda i: (i, 0)),
            pl.BlockSpec((_TB, _D), lambda i: (i, 0)),
            pl.BlockSpec((_TB, 1), lambda i: (i, 0)),
            pl.BlockSpec((_D, _L), lambda i: (0, 0)),
            pl.BlockSpec((_L, _D), lambda i: (0, 0)),
            pl.BlockSpec((_D, _C), lambda i: (0, 0)),
        ],
        out_specs=pl.BlockSpec((1, 1, 128), lambda i: (i, 0, 0)),
        out_shape=jax.ShapeDtypeStruct((_GRID, 1, 128), jnp.float32),
        compiler_params=pltpu.CompilerParams(
            dimension_semantics=("arbitrary",)),
    )(enc16, decoder_input, lab2, wenc16, wdz16, wout16)

    return (jnp.sum(partials[:, 0, 0]) / _B).reshape(())
